# baseline (speedup 1.0000x reference)
import functools

import jax
import jax.numpy as jnp
from jax import lax
from jax.experimental import pallas as pl
from jax.experimental.pallas import tpu as pltpu
from jax.experimental.pallas import tpu_sc as plsc

ROWS, N = 128, 32768
NC, NS = 2, 16
NW = NC * NS
RPW = ROWS // NW
LANES = 16
NVEC = N // LANES
NPASS = 8
FMASK = 0x7FFFFFFF

_GDN = jax.lax.GatherDimensionNumbers(
    offset_dims=(), collapsed_slice_dims=(0,), start_index_map=(0,)
)


def _vperm(v, idx):
    return jax.lax.gather(
        v, idx[:, None], _GDN, slice_sizes=(1,),
        mode=jax.lax.GatherScatterMode.PROMISE_IN_BOUNDS,
    )


def _sort_body(in_hbm, out_hbm, bufa, bufb, hist):
    wid = lax.axis_index("s") * NC + lax.axis_index("c")
    lane = lax.iota(jnp.int32, LANES)
    zeros = jnp.zeros((LANES,), jnp.int32)

    def fwd_key(u):
        return jnp.where(u < 0, u, u ^ FMASK)

    def one_pass(src, dst, shift, first, last_pass):
        def digits(i):
            v = src[pl.ds(i * LANES, LANES)]
            key = fwd_key(v) if first else v
            if shift == 0:
                d = key & 0xF
            elif shift == 32 - 4:
                d = lax.shift_right_logical(key, shift)
            else:
                d = lax.shift_right_logical(key, shift) & 0xF
            return key, d

        @plsc.parallel_loop(0, NVEC, unroll=32, carry=zeros)
        def tot(i, acc):
            _, d = digits(i)
            occ, lastm = plsc.scan_count(d)
            hist[pl.ds(i * LANES, LANES)] = zeros
            plsc.store_scatter(hist, [i * LANES + d], occ, mask=lastm)
            return acc + hist[pl.ds(i * LANES, LANES)]

        base = plsc.cumsum(tot) - tot - 1

        @plsc.parallel_loop(0, NVEC, unroll=32, carry=base)
        def _run(i, run):
            key, d = digits(i)
            h = hist[pl.ds(i * LANES, LANES)]
            occ, _ = plsc.scan_count(d)
            pos = _vperm(run, d) + occ
            out = jnp.where(key < 0, key, key ^ FMASK) if last_pass else key
            plsc.store_scatter(dst, [pos], out)
            return run + h

    def row_body(r, _):
        row = wid * RPW + r
        pltpu.sync_copy(in_hbm.at[row], bufa)
        for p in range(NPASS):
            src, dst = (bufa, bufb) if p % 2 == 0 else (bufb, bufa)
            one_pass(src, dst, p * 4, first=(p == 0), last_pass=(p == NPASS - 1))
        pltpu.sync_copy(bufa, out_hbm.at[row])
        return 0

    lax.fori_loop(0, RPW, row_body, 0)


@functools.partial(
    pl.kernel,
    out_type=jax.ShapeDtypeStruct((ROWS, N), jnp.int32),
    mesh=plsc.VectorSubcoreMesh(core_axis_name="c", subcore_axis_name="s"),
    scratch_types=[
        pltpu.VMEM((N,), jnp.int32),
        pltpu.VMEM((N,), jnp.int32),
        pltpu.VMEM((N,), jnp.int32),
    ],
    compiler_params=pltpu.CompilerParams(needs_layout_passes=False),
)
def _sort_kernel(in_hbm, out_hbm, bufa, bufb, hist):
    _sort_body(in_hbm, out_hbm, bufa, bufb, hist)


def kernel(inputs):
    raw = jax.lax.bitcast_convert_type(inputs, jnp.int32)
    out = _sort_kernel(raw)
    return jax.lax.bitcast_convert_type(out, jnp.float32)


# speedup vs baseline: 1.0803x; 1.0803x over previous
import functools

import jax
import jax.numpy as jnp
from jax import lax
from jax.experimental import pallas as pl
from jax.experimental.pallas import tpu as pltpu
from jax.experimental.pallas import tpu_sc as plsc

ROWS, N = 128, 32768
NC, NS = 2, 16
NW = NC * NS
RPW = ROWS // NW
LANES = 16
NVEC = N // LANES
NPASS = 8
FMASK = 0x7FFFFFFF

_GDN = jax.lax.GatherDimensionNumbers(
    offset_dims=(), collapsed_slice_dims=(0,), start_index_map=(0,)
)


def _vperm(v, idx):
    return jax.lax.gather(
        v, idx[:, None], _GDN, slice_sizes=(1,),
        mode=jax.lax.GatherScatterMode.PROMISE_IN_BOUNDS,
    )


def _sort_body(in_hbm, out_hbm, bufa, bufb, hist):
    wid = lax.axis_index("s") * NC + lax.axis_index("c")
    lane = lax.iota(jnp.int32, LANES)
    zeros = jnp.zeros((LANES,), jnp.int32)

    def fwd_key(u):
        return jnp.where(u < 0, u, u ^ FMASK)

    def one_pass(src, dst, shift, first, last_pass):
        def digits(i):
            v = src[pl.ds(i * LANES, LANES)]
            key = fwd_key(v) if first else v
            if shift == 0:
                d = key & 0xF
            elif shift == 32 - 4:
                d = lax.shift_right_logical(key, shift)
            else:
                d = lax.shift_right_logical(key, shift) & 0xF
            return key, d

        @plsc.parallel_loop(0, NVEC, unroll=16)
        def _hist(i):
            _, d = digits(i)
            occ, lastm = plsc.scan_count(d)
            hist[pl.ds(i * LANES, LANES)] = zeros
            plsc.store_scatter(hist, [i * LANES + d], occ, mask=lastm)

        @plsc.parallel_loop(0, NVEC, unroll=16, carry=zeros)
        def tot(i, acc):
            return acc + hist[pl.ds(i * LANES, LANES)]

        base = plsc.cumsum(tot) - tot - 1

        @plsc.parallel_loop(0, NVEC, unroll=16, carry=base)
        def _run(i, run):
            key, d = digits(i)
            h = hist[pl.ds(i * LANES, LANES)]
            occ, _ = plsc.scan_count(d)
            pos = _vperm(run, d) + occ
            out = jnp.where(key < 0, key, key ^ FMASK) if last_pass else key
            plsc.store_scatter(dst, [pos], out)
            return run + h

    def row_body(r, _):
        row = wid * RPW + r
        pltpu.sync_copy(in_hbm.at[row], bufa)
        for p in range(NPASS):
            src, dst = (bufa, bufb) if p % 2 == 0 else (bufb, bufa)
            one_pass(src, dst, p * 4, first=(p == 0), last_pass=(p == NPASS - 1))
        pltpu.sync_copy(bufa, out_hbm.at[row])
        return 0

    lax.fori_loop(0, RPW, row_body, 0)


@functools.partial(
    pl.kernel,
    out_type=jax.ShapeDtypeStruct((ROWS, N), jnp.int32),
    mesh=plsc.VectorSubcoreMesh(core_axis_name="c", subcore_axis_name="s"),
    scratch_types=[
        pltpu.VMEM((N,), jnp.int32),
        pltpu.VMEM((N,), jnp.int32),
        pltpu.VMEM((N,), jnp.int32),
    ],
    compiler_params=pltpu.CompilerParams(needs_layout_passes=False),
)
def _sort_kernel(in_hbm, out_hbm, bufa, bufb, hist):
    _sort_body(in_hbm, out_hbm, bufa, bufb, hist)


def kernel(inputs):
    raw = jax.lax.bitcast_convert_type(inputs, jnp.int32)
    out = _sort_kernel(raw)
    return jax.lax.bitcast_convert_type(out, jnp.float32)


# speedup vs baseline: 1.2146x; 1.1243x over previous
import functools

import jax
import jax.numpy as jnp
from jax import lax
from jax.experimental import pallas as pl
from jax.experimental.pallas import tpu as pltpu
from jax.experimental.pallas import tpu_sc as plsc

ROWS, N = 128, 32768
NC, NS = 2, 16
NW = NC * NS
RPW = ROWS // NW
LANES = 16
NVEC = N // LANES
NPASS = 8
FMASK = 0x7FFFFFFF

_GDN = jax.lax.GatherDimensionNumbers(
    offset_dims=(), collapsed_slice_dims=(0,), start_index_map=(0,)
)


def _vperm(v, idx):
    return jax.lax.gather(
        v, idx[:, None], _GDN, slice_sizes=(1,),
        mode=jax.lax.GatherScatterMode.PROMISE_IN_BOUNDS,
    )


def _sort_body(in_hbm, out_hbm, bufa, bufb, hist):
    wid = lax.axis_index("s") * NC + lax.axis_index("c")
    zeros = jnp.zeros((LANES,), jnp.int32)

    def fwd_key(u):
        return jnp.where(u < 0, u, u ^ FMASK)

    def one_pass(src, dst, shift, first, last_pass):
        def digits(i):
            v = src[pl.ds(i * LANES, LANES)]
            key = fwd_key(v) if first else v
            if shift == 0:
                d = key & 0xF
            elif shift == 32 - 4:
                d = lax.shift_right_logical(key, shift)
            else:
                d = lax.shift_right_logical(key, shift) & 0xF
            return key, d

        @plsc.parallel_loop(0, NVEC, unroll=16, carry=zeros)
        def tot(i, acc):
            _, d = digits(i)
            occ, lastm = plsc.scan_count(d)
            hist[pl.ds(i * LANES, LANES)] = zeros
            plsc.store_scatter(hist, [i * LANES + d], occ, mask=lastm)
            return acc + hist[pl.ds(i * LANES, LANES)]

        base = plsc.cumsum(tot) - tot - 1

        @plsc.parallel_loop(0, NVEC, unroll=16, carry=base)
        def _run(i, run):
            key, d = digits(i)
            h = hist[pl.ds(i * LANES, LANES)]
            occ, _ = plsc.scan_count(d)
            pos = _vperm(run, d) + occ
            out = jnp.where(key < 0, key, key ^ FMASK) if last_pass else key
            plsc.store_scatter(dst, [pos], out)
            return run + h

    def row_body(r, _):
        row = wid * RPW + r
        pltpu.sync_copy(in_hbm.at[row], bufa)
        for p in range(NPASS):
            src, dst = (bufa, bufb) if p % 2 == 0 else (bufb, bufa)
            one_pass(src, dst, p * 4, first=(p == 0), last_pass=(p == NPASS - 1))
        pltpu.sync_copy(bufa, out_hbm.at[row])
        return 0

    lax.fori_loop(0, RPW, row_body, 0)


@functools.partial(
    pl.kernel,
    out_type=jax.ShapeDtypeStruct((ROWS, N), jnp.int32),
    mesh=plsc.VectorSubcoreMesh(core_axis_name="c", subcore_axis_name="s"),
    scratch_types=[
        pltpu.VMEM((N,), jnp.int32),
        pltpu.VMEM((N,), jnp.int32),
        pltpu.VMEM((N,), jnp.int32),
    ],
    compiler_params=pltpu.CompilerParams(needs_layout_passes=False),
)
def _sort_kernel(in_hbm, out_hbm, bufa, bufb, hist):
    _sort_body(in_hbm, out_hbm, bufa, bufb, hist)


def kernel(inputs):
    raw = jax.lax.bitcast_convert_type(inputs, jnp.int32)
    out = _sort_kernel(raw)
    return jax.lax.bitcast_convert_type(out, jnp.float32)
